# baseline (speedup 1.0000x reference)
import jax
import jax.numpy as jnp
from jax.experimental import pallas as pl

N = 2097152
N_VALUES = 64
BLOCK = 16384


def _onehot_block(x_ref, o_ref):
    idx = x_ref[...]
    iota = jax.lax.broadcasted_iota(jnp.int32, (BLOCK, N_VALUES), 1)
    o_ref[...] = (idx == iota).astype(jnp.float32)


def kernel(x):
    x2 = x.astype(jnp.int32).reshape(N, 1)
    return pl.pallas_call(
        _onehot_block,
        grid=(N // BLOCK,),
        in_specs=[pl.BlockSpec((BLOCK, 1), lambda i: (i, 0))],
        out_specs=pl.BlockSpec((BLOCK, N_VALUES), lambda i: (i, 0)),
        out_shape=jax.ShapeDtypeStruct((N, N_VALUES), jnp.float32),
    )(x2)


# speedup vs baseline: 1.2760x; 1.2760x over previous
import dataclasses
import functools

import jax
import jax.numpy as jnp
from jax import lax
from jax.experimental import pallas as pl
from jax.experimental.pallas import tpu as pltpu
from jax.experimental.pallas import tpu_sc as plsc

N = 2097152
NV = 64
NW = 32
PER_W = N // NW
W = 512
NWIN = PER_W // W
L = 16

_mesh = plsc.VectorSubcoreMesh(core_axis_name="c", subcore_axis_name="s")

_cp = pltpu.CompilerParams(
    needs_layout_passes=False, use_tc_tiling_on_sc=False
)


@functools.partial(
    pl.kernel,
    mesh=_mesh,
    compiler_params=_cp,
    out_type=jax.ShapeDtypeStruct((N, NV), jnp.float32),
    scratch_types=[
        pltpu.VMEM((W,), jnp.int32),
        pltpu.VMEM((W,), jnp.int32),
        pltpu.VMEM((W,), jnp.int32),
        pltpu.VMEM((W,), jnp.int32),
        pltpu.VMEM((W, NV), jnp.float32),
        pltpu.VMEM((W, NV), jnp.float32),
        pltpu.SemaphoreType.DMA,
        pltpu.SemaphoreType.DMA,
    ],
)
def _sc_onehot(
    x_hbm, out_hbm, idx0, idx1, col0, col1, rows0, rows1, in_sem, out_sem
):
    idx_v = (idx0, idx1)
    col_v = (col0, col1)
    rows_v = (rows0, rows1)
    wid = lax.axis_index("c") * 16 + lax.axis_index("s")
    base = wid * PER_W

    iota = lax.iota(jnp.int32, L)
    ones = jnp.full((L,), 1.0, jnp.float32)
    zeros = jnp.zeros((L,), jnp.float32)

    for b in range(2):
        rows = rows_v[b]

        @pl.loop(0, W)
        def _(r):
            for c in range(0, NV, L):
                rows[r, pl.ds(c, L)] = zeros

    for b in range(2):
        pltpu.make_async_copy(
            x_hbm.at[pl.ds(base + b * W, W)], idx_v[b], in_sem
        ).start()

    @pl.loop(0, NWIN, step=2)
    def _(tt):
        for b in range(2):
            t = tt + b
            rows = rows_v[b]
            idx = idx_v[b]
            col = col_v[b]

            @pl.when(t >= 2)
            def _():
                pltpu.make_async_copy(
                    rows, out_hbm.at[pl.ds(0, W)], out_sem
                ).wait()

                @pl.loop(0, W, step=L)
                def _(k):
                    old = col[pl.ds(k, L)]
                    plsc.store_scatter(rows, [k + iota, old], zeros)

            pltpu.make_async_copy(
                x_hbm.at[pl.ds(base, W)], idx, in_sem
            ).wait()

            @pl.loop(0, W, step=L)
            def _(k):
                vvec = idx[pl.ds(k, L)]
                plsc.store_scatter(rows, [k + iota, vvec], ones)
                col[pl.ds(k, L)] = vvec

            pltpu.make_async_copy(
                rows, out_hbm.at[pl.ds(base + t * W, W)], out_sem
            ).start()

            @pl.when(t + 2 < NWIN)
            def _():
                pltpu.make_async_copy(
                    x_hbm.at[pl.ds(base + (t + 2) * W, W)],
                    idx_v[b],
                    in_sem,
                ).start()

    for b in range(2):
        pltpu.make_async_copy(
            rows_v[b], out_hbm.at[pl.ds(0, W)], out_sem
        ).wait()


def kernel(x):
    return _sc_onehot(x.astype(jnp.int32))


# speedup vs baseline: 9.9090x; 7.7654x over previous
import jax
import jax.numpy as jnp
from jax.experimental import pallas as pl

N = 2097152
NV = 64
C = 16384
G = N // C


def _onehot_t(x_ref, o_ref):
    i = pl.program_id(0)
    xb = x_ref[0, i % 8]
    cls = jax.lax.broadcasted_iota(jnp.int32, (NV, C), 0)
    o_ref[...] = (xb[None, :] == cls).astype(jnp.float32)


def kernel(x):
    x3 = x.astype(jnp.int32).reshape(G // 8, 8, C)
    out_t = pl.pallas_call(
        _onehot_t,
        grid=(G,),
        in_specs=[pl.BlockSpec((1, 8, C), lambda i: (i // 8, 0, 0))],
        out_specs=pl.BlockSpec((NV, C), lambda i: (0, i)),
        out_shape=jax.ShapeDtypeStruct((NV, N), jnp.float32),
    )(x3)
    return out_t.T


# speedup vs baseline: 10.5729x; 1.0670x over previous
import jax
import jax.numpy as jnp
from jax.experimental import pallas as pl

N = 2097152
NV = 64
C = 32768
G = N // C


def _onehot_t(x_ref, o_ref):
    i = pl.program_id(0)
    xb = x_ref[0, i % 8]
    cls = jax.lax.broadcasted_iota(jnp.int32, (NV, C), 0)
    o_ref[...] = (xb[None, :] == cls).astype(jnp.float32)


def kernel(x):
    x3 = x.astype(jnp.int32).reshape(G // 8, 8, C)
    out_t = pl.pallas_call(
        _onehot_t,
        grid=(G,),
        in_specs=[pl.BlockSpec((1, 8, C), lambda i: (i // 8, 0, 0))],
        out_specs=pl.BlockSpec((NV, C), lambda i: (0, i)),
        out_shape=jax.ShapeDtypeStruct((NV, N), jnp.float32),
    )(x3)
    return out_t.T


# speedup vs baseline: 10.5990x; 1.0025x over previous
import jax
import jax.numpy as jnp
from jax.experimental import pallas as pl

N = 2097152
NV = 64
C = 65536
G = N // C


def _onehot_t(x_ref, o_ref):
    i = pl.program_id(0)
    xb = x_ref[0, i % 8]
    cls = jax.lax.broadcasted_iota(jnp.int32, (NV, C), 0)
    o_ref[...] = (xb[None, :] == cls).astype(jnp.float32)


def kernel(x):
    x3 = x.astype(jnp.int32).reshape(G // 8, 8, C)
    out_t = pl.pallas_call(
        _onehot_t,
        grid=(G,),
        in_specs=[pl.BlockSpec((1, 8, C), lambda i: (i // 8, 0, 0))],
        out_specs=pl.BlockSpec((NV, C), lambda i: (0, i)),
        out_shape=jax.ShapeDtypeStruct((NV, N), jnp.float32),
    )(x3)
    return out_t.T
